# baseline (speedup 1.0000x reference)
import functools

import jax
import jax.numpy as jnp
from jax import lax
from jax.experimental import pallas as pl
from jax.experimental.pallas import tpu as pltpu
from jax.experimental.pallas import tpu_sc as plsc

N_ATOMS = 10000
N_MOL = 500
H = 128
BOND_DIM = 16
DEPTH = 3

HALF = 80000
HALF_PAD = 81920
E_PAD = 2 * HALF_PAD
PAD = HALF_PAD - HALF

NC, NS = 2, 16
NW = NC * NS
PER_W = E_PAD // NW
CHUNK = 128
N_CHUNKS = PER_W // CHUNK
SUP = 256
SPC = SUP // CHUNK
N_SUP = PER_W // SUP
SUP_S = CHUNK
N_SUP_S = PER_W // SUP_S

PER_S = E_PAD // NS
N_CHUNKS_S = PER_S // CHUNK
N_A_PAD = 10240
ROWS_PER_SUB = N_A_PAD // NS
ZCHUNK = 64

BLK = 2048
NBLK = E_PAD // BLK
BLKA = 1000
NBLKA = N_ATOMS // BLKA

_HIGHEST = lax.Precision.HIGHEST


def _worker_id():
    return lax.axis_index("c") * NS + lax.axis_index("s")


_sc_cache = {}


def _sc_kernels():
    if _sc_cache:
        return _sc_cache["g1"], _sc_cache["msg"], _sc_cache["scat"]

    mesh = plsc.VectorSubcoreMesh(core_axis_name="c", subcore_axis_name="s",
                                  num_cores=NC, num_subcores=NS)

    @functools.partial(
        pl.kernel,
        out_type=jax.ShapeDtypeStruct((E_PAD, H), jnp.float32),
        mesh=mesh,
        scratch_types=[
            pltpu.VMEM((N_CHUNKS, CHUNK), jnp.int32),
            pltpu.VMEM((CHUNK, H), jnp.float32),
            pltpu.VMEM((CHUNK, H), jnp.float32),
            pltpu.VMEM_SHARED((N_A_PAD, H), jnp.float32),
            pltpu.SemaphoreType.DMA,
            pltpu.SemaphoreType.DMA,
            pltpu.SemaphoreType.DMA,
            pltpu.SemaphoreType.DMA,
        ],
    )
    def _sc_gather1(tab_hbm, idx_hbm, out_hbm, idx_v, b0, b1, tab_sh,
                    gs0, gs1, os0, os1):
        s = lax.axis_index("s")
        wid = lax.axis_index("c") * NS + lax.axis_index("s")
        base = wid * PER_W
        pltpu.sync_copy(idx_hbm.at[wid], idx_v)

        def stage(k, carry):
            r = s * ROWS_PER_SUB + k * CHUNK
            pltpu.sync_copy(tab_hbm.at[pl.ds(r, CHUNK)], b0)
            pltpu.sync_copy(b0, tab_sh.at[pl.ds(r, CHUNK)])
            return carry

        lax.fori_loop(0, ROWS_PER_SUB // CHUNK, stage, None)
        plsc.subcore_barrier()

        def fire_g(j, buf, sem):
            pltpu.async_copy(tab_sh.at[idx_v.at[j]], buf, sem)

        def drain_g(buf, sem):
            pltpu.make_async_copy(tab_sh.at[idx_v.at[0]], buf, sem).wait()

        def fire_out(j, buf, sem):
            pltpu.async_copy(buf, out_hbm.at[pl.ds(base + j * CHUNK, CHUNK)],
                             sem)

        def wait_out(buf, sem):
            pltpu.make_async_copy(buf, out_hbm.at[pl.ds(base, CHUNK)],
                                  sem).wait()

        fire_g(0, b0, gs0)

        def body(jj, carry):
            j0 = 2 * jj
            drain_g(b0, gs0)

            @pl.when(jj > 0)
            def _():
                wait_out(b1, os1)

            fire_g(j0 + 1, b1, gs1)
            fire_out(j0, b0, os0)
            drain_g(b1, gs1)
            wait_out(b0, os0)

            @pl.when(jj < N_CHUNKS // 2 - 1)
            def _():
                fire_g(j0 + 2, b0, gs0)

            fire_out(j0 + 1, b1, os1)
            return carry

        lax.fori_loop(0, N_CHUNKS // 2, body, None)
        wait_out(b1, os1)

    @functools.partial(
        pl.kernel,
        out_type=jax.ShapeDtypeStruct((E_PAD, H), jnp.float32),
        mesh=mesh,
        scratch_types=[
            pltpu.VMEM((N_CHUNKS_S, CHUNK), jnp.int32),
            pltpu.VMEM((N_CHUNKS, CHUNK), jnp.int32),
            pltpu.VMEM((CHUNK, H), jnp.float32),
            pltpu.VMEM((CHUNK, H), jnp.float32),
            pltpu.VMEM_SHARED((N_A_PAD, H), jnp.float32),
            pltpu.SemaphoreType.DMA,
            pltpu.SemaphoreType.DMA,
            pltpu.SemaphoreType.DMA,
            pltpu.SemaphoreType.DMA,
        ],
    )
    def _sc_msg(h_hbm, idxd_hbm, idxs_hbm, g_hbm, idxd_v, idxs_v, b0, b1,
                agg_sh, sa0, sa1, sb0, sb1):
        s = lax.axis_index("s")
        wid = lax.axis_index("c") * NS + s
        sbase = s * PER_S
        gbase = wid * PER_W

        def zrow(i, carry):
            for cc in range(H // 16):
                b0[i, pl.ds(cc * 16, 16)] = jnp.zeros((16,), jnp.float32)
            return carry

        lax.fori_loop(0, CHUNK, zrow, None)

        def zcp(k, carry):
            r = s * ROWS_PER_SUB + k * CHUNK
            pltpu.sync_copy(b0, agg_sh.at[pl.ds(r, CHUNK)])
            return carry

        lax.fori_loop(0, ROWS_PER_SUB // CHUNK, zcp, None)
        pltpu.sync_copy(idxd_hbm.at[s], idxd_v)
        pltpu.sync_copy(idxs_hbm.at[wid], idxs_v)
        plsc.subcore_barrier()

        def fire_load(j, buf, sem):
            pltpu.async_copy(h_hbm.at[pl.ds(sbase + j * CHUNK, CHUNK)],
                             buf, sem)

        def wait_load(buf, sem):
            pltpu.make_async_copy(h_hbm.at[pl.ds(sbase, CHUNK)], buf,
                                  sem).wait()

        def fire_add(j, buf, sem):
            pltpu.async_copy(buf, agg_sh.at[idxd_v.at[j]], sem, add=True)

        def drain_add(buf, sem):
            pltpu.make_async_copy(buf, agg_sh.at[idxd_v.at[0]], sem).wait()

        fire_load(0, b0, sa0)

        def sup(jj, carry):
            j0 = 2 * jj
            wait_load(b0, sa0)
            fire_add(j0, b0, sb0)

            @pl.when(jj > 0)
            def _():
                drain_add(b1, sb1)

            fire_load(j0 + 1, b1, sa1)
            drain_add(b0, sb0)

            @pl.when(jj < N_CHUNKS_S // 2 - 1)
            def _():
                fire_load(j0 + 2, b0, sa0)

            wait_load(b1, sa1)
            fire_add(j0 + 1, b1, sb1)
            return carry

        lax.fori_loop(0, N_CHUNKS_S // 2, sup, None)
        drain_add(b1, sb1)
        plsc.subcore_barrier()

        def fire_g(j, buf, sem):
            pltpu.async_copy(agg_sh.at[idxs_v.at[j]], buf, sem)

        def drain_g(buf, sem):
            pltpu.make_async_copy(agg_sh.at[idxs_v.at[0]], buf, sem).wait()

        def fire_out(j, buf, sem):
            pltpu.async_copy(buf, g_hbm.at[pl.ds(gbase + j * CHUNK, CHUNK)],
                             sem)

        def wait_out(buf, sem):
            pltpu.make_async_copy(buf, g_hbm.at[pl.ds(gbase, CHUNK)],
                                  sem).wait()

        fire_g(0, b0, sa0)

        def body(jj, carry):
            j0 = 2 * jj
            drain_g(b0, sa0)

            @pl.when(jj > 0)
            def _():
                wait_out(b1, sb1)

            fire_g(j0 + 1, b1, sa1)
            fire_out(j0, b0, sb0)
            drain_g(b1, sa1)
            wait_out(b0, sb0)

            @pl.when(jj < N_CHUNKS // 2 - 1)
            def _():
                fire_g(j0 + 2, b0, sa0)

            fire_out(j0 + 1, b1, sb1)
            return carry

        lax.fori_loop(0, N_CHUNKS // 2, body, None)
        wait_out(b1, sb1)

    @functools.partial(
        pl.kernel,
        out_type=jax.ShapeDtypeStruct((2 * N_A_PAD, H), jnp.float32),
        mesh=mesh,
        scratch_types=[
            pltpu.VMEM((N_CHUNKS, CHUNK), jnp.int32),
            pltpu.VMEM((SUP_S, H), jnp.float32),
            pltpu.VMEM((SUP_S, H), jnp.float32),
            pltpu.VMEM((ZCHUNK, H), jnp.float32),
            pltpu.VMEM_SHARED((N_A_PAD, H), jnp.float32),
            pltpu.SemaphoreType.DMA,
            pltpu.SemaphoreType.DMA,
            pltpu.SemaphoreType.DMA,
            pltpu.SemaphoreType.DMA,
        ],
    )
    def _sc_scatter(rows_hbm, idx_hbm, out_hbm, idx_v, b0, b1, zbuf_v,
                    agg_sh, ls0, ls1, ss0, ss1):
        c = lax.axis_index("c")
        s = lax.axis_index("s")
        wid = c * NS + s
        base = wid * PER_W

        def zrow(i, carry):
            for cc in range(H // 16):
                zbuf_v[i, pl.ds(cc * 16, 16)] = jnp.zeros((16,), jnp.float32)
            return carry

        lax.fori_loop(0, ZCHUNK, zrow, None)

        def zcp(k, carry):
            r = s * ROWS_PER_SUB + k * ZCHUNK
            pltpu.sync_copy(zbuf_v, agg_sh.at[pl.ds(r, ZCHUNK)])
            return carry

        lax.fori_loop(0, ROWS_PER_SUB // ZCHUNK, zcp, None)
        plsc.subcore_barrier()

        pltpu.sync_copy(idx_hbm.at[wid], idx_v)

        def fire_load(j, buf, sem):
            pltpu.async_copy(rows_hbm.at[pl.ds(base + j * SUP_S, SUP_S)],
                             buf, sem)

        def wait_load(buf, sem):
            pltpu.make_async_copy(rows_hbm.at[pl.ds(base, SUP_S)], buf,
                                  sem).wait()

        def fire_add(j, buf, sem):
            pltpu.async_copy(buf, agg_sh.at[idx_v.at[j]], sem, add=True)

        def drain_add(buf, sem):
            pltpu.make_async_copy(buf, agg_sh.at[idx_v.at[0]], sem).wait()

        fire_load(0, b0, ls0)

        def sup(jj, carry):
            j0 = 2 * jj
            wait_load(b0, ls0)
            fire_add(j0, b0, ss0)

            @pl.when(jj > 0)
            def _():
                drain_add(b1, ss1)

            fire_load(j0 + 1, b1, ls1)
            drain_add(b0, ss0)

            @pl.when(jj < N_SUP_S // 2 - 1)
            def _():
                fire_load(j0 + 2, b0, ls0)

            wait_load(b1, ls1)
            fire_add(j0 + 1, b1, ss1)
            return carry

        lax.fori_loop(0, N_SUP_S // 2, sup, None)
        drain_add(b1, ss1)
        plsc.subcore_barrier()

        def cp(k, carry):
            r = s * ROWS_PER_SUB + k * ZCHUNK
            pltpu.sync_copy(agg_sh.at[pl.ds(r, ZCHUNK)], zbuf_v)
            pltpu.sync_copy(zbuf_v, out_hbm.at[pl.ds(c * N_A_PAD + r, ZCHUNK)])
            return carry

        lax.fori_loop(0, ROWS_PER_SUB // ZCHUNK, cp, None)

    _sc_cache.update(g1=_sc_gather1, msg=_sc_msg, scat=_sc_scatter)
    return _sc_cache["g1"], _sc_cache["msg"], _sc_cache["scat"]


def _pad_mask(i, blk):
    row = i * blk + lax.broadcasted_iota(jnp.int32, (blk, H), 0)
    return (row % HALF_PAD) < HALF


def _init_body(bond_ref, xg_ref, wb_ref, wx_ref, b_ref, out_ref):
    acc = jnp.dot(bond_ref[...], wb_ref[...],
                  preferred_element_type=jnp.float32, precision=_HIGHEST)
    acc += jnp.dot(xg_ref[...], wx_ref[...],
                   preferred_element_type=jnp.float32, precision=_HIGHEST)
    h0 = jnp.maximum(acc + b_ref[...], 0.0)
    out_ref[...] = jnp.where(_pad_mask(pl.program_id(0), BLK), h0, 0.0)


_tc_init = pl.pallas_call(
    _init_body,
    grid=(NBLK,),
    in_specs=[
        pl.BlockSpec((BLK, BOND_DIM), lambda i: (i, 0)),
        pl.BlockSpec((BLK, H), lambda i: (i, 0)),
        pl.BlockSpec((BOND_DIM, H), lambda i: (0, 0)),
        pl.BlockSpec((H, H), lambda i: (0, 0)),
        pl.BlockSpec((1, H), lambda i: (0, 0)),
    ],
    out_specs=pl.BlockSpec((BLK, H), lambda i: (i, 0)),
    out_shape=jax.ShapeDtypeStruct((E_PAD, H), jnp.float32),
)


def _combine_body(a_ref, b_ref, out_ref):
    out_ref[...] = a_ref[...] + b_ref[...]


_tc_combine = pl.pallas_call(
    _combine_body,
    grid=(N_A_PAD // 1024,),
    in_specs=[
        pl.BlockSpec((1024, H), lambda i: (i, 0)),
        pl.BlockSpec((1024, H), lambda i: (i, 0)),
    ],
    out_specs=pl.BlockSpec((1024, H), lambda i: (i, 0)),
    out_shape=jax.ShapeDtypeStruct((N_A_PAD, H), jnp.float32),
)


def _depth_body(h0_ref, g_ref, hrev_ref, w_ref, b_ref, out_ref):
    m = g_ref[...] - hrev_ref[...]
    acc = jnp.dot(m, w_ref[...],
                  preferred_element_type=jnp.float32, precision=_HIGHEST)
    h = jnp.maximum(h0_ref[...] + acc + b_ref[...], 0.0)
    out_ref[...] = jnp.where(_pad_mask(pl.program_id(0), BLK), h, 0.0)


_tc_depth = pl.pallas_call(
    _depth_body,
    grid=(NBLK,),
    in_specs=[
        pl.BlockSpec((BLK, H), lambda i: (i, 0)),
        pl.BlockSpec((BLK, H), lambda i: (i, 0)),
        pl.BlockSpec((BLK, H), lambda i: ((i + NBLK // 2) % NBLK, 0)),
        pl.BlockSpec((H, H), lambda i: (0, 0)),
        pl.BlockSpec((1, H), lambda i: (0, 0)),
    ],
    out_specs=pl.BlockSpec((BLK, H), lambda i: (i, 0)),
    out_shape=jax.ShapeDtypeStruct((E_PAD, H), jnp.float32),
)


def _final_body(atom_ref, agga_ref, aggb_ref, a2m_ref, molf_ref,
                wx_ref, wm_ref, b_ref, out_ref):
    i = pl.program_id(0)
    mv = agga_ref[...] + aggb_ref[...]
    hv = jnp.dot(atom_ref[...], wx_ref[...],
                 preferred_element_type=jnp.float32, precision=_HIGHEST)
    hv += jnp.dot(mv, wm_ref[...],
                  preferred_element_type=jnp.float32, precision=_HIGHEST)
    hv = jnp.maximum(hv + b_ref[...], 0.0)
    seg = a2m_ref[0, 0, :]
    onehot = (lax.broadcasted_iota(jnp.int32, (N_MOL, BLKA), 0)
              == seg[None, :]).astype(jnp.float32)
    contrib = jnp.dot(onehot, hv,
                      preferred_element_type=jnp.float32, precision=_HIGHEST)

    @pl.when(i == 0)
    def _():
        out_ref[:, :H] = contrib
        out_ref[:, H:] = molf_ref[...]

    @pl.when(i != 0)
    def _():
        out_ref[:, :H] = out_ref[:, :H] + contrib


_tc_final = pl.pallas_call(
    _final_body,
    grid=(NBLKA,),
    in_specs=[
        pl.BlockSpec((BLKA, H), lambda i: (i, 0)),
        pl.BlockSpec((BLKA, H), lambda i: (i, 0)),
        pl.BlockSpec((BLKA, H), lambda i: (i, 0)),
        pl.BlockSpec((1, 1, BLKA), lambda i: (i, 0, 0)),
        pl.BlockSpec((N_MOL, H), lambda i: (0, 0)),
        pl.BlockSpec((H, H), lambda i: (0, 0)),
        pl.BlockSpec((H, H), lambda i: (0, 0)),
        pl.BlockSpec((1, H), lambda i: (0, 0)),
    ],
    out_specs=pl.BlockSpec((N_MOL, 2 * H), lambda i: (0, 0)),
    out_shape=jax.ShapeDtypeStruct((N_MOL, 2 * H), jnp.float32),
)


def _mid_pad(x):
    z = jnp.zeros((PAD,) + x.shape[1:], dtype=x.dtype)
    return jnp.concatenate([x[:HALF], z, x[HALF:], z], axis=0)


def _mid_pad_idx(x):
    z = jnp.arange(PAD, dtype=x.dtype) % N_ATOMS
    return jnp.concatenate([x[:HALF], z, x[HALF:], z], axis=0)


def kernel(atom_features, bond_features, molecule_features, Wi_w, Wi_b,
           Wm_w, Wm_b, Wa_w, Wa_b, bond_index, atom_to_molecule, b2rev):
    src = bond_index[0].astype(jnp.int32)
    dst = bond_index[1].astype(jnp.int32)

    src_p = _mid_pad_idx(src)
    dst_p = _mid_pad_idx(dst)
    bond_p = _mid_pad(bond_features)

    src_t = src_p.reshape(NW, N_CHUNKS, CHUNK)
    dst_t = dst_p.reshape(NW, N_CHUNKS, CHUNK)

    wi_t = Wi_w.T
    wb_t = wi_t[:BOND_DIM]
    wx_t = wi_t[BOND_DIM:]
    wm_t = Wm_w.T
    wa_t = Wa_w.T
    wax_t = wa_t[:H]
    wam_t = wa_t[H:]

    wi_b = Wi_b.reshape(1, H)
    wm_b = Wm_b.reshape(1, H)
    wa_b = Wa_b.reshape(1, H)

    a2m_t = atom_to_molecule.astype(jnp.int32).reshape(NBLKA, 1, BLKA)

    sc_gather1, sc_msg, sc_scatter = _sc_kernels()

    xg = sc_gather1(atom_features, src_t)
    h0 = _tc_init(bond_p, xg, wb_t, wx_t, wi_b)

    h = h0
    dst_s = dst_p.reshape(NS, N_CHUNKS_S, CHUNK)
    for _ in range(DEPTH):
        g = sc_msg(h, dst_s, src_t)
        h = _tc_depth(h0, g, h, wm_t, wm_b)

    agg2f = sc_scatter(h, src_t)
    return _tc_final(atom_features, agg2f[:N_ATOMS], agg2f[N_A_PAD:N_A_PAD + N_ATOMS],
                     a2m_t, molecule_features, wax_t, wam_t, wa_b)


# speedup vs baseline: 1.0657x; 1.0657x over previous
import functools

import jax
import jax.numpy as jnp
from jax import lax
from jax.experimental import pallas as pl
from jax.experimental.pallas import tpu as pltpu
from jax.experimental.pallas import tpu_sc as plsc

N_ATOMS = 10000
N_MOL = 500
H = 128
BOND_DIM = 16
DEPTH = 3

HALF = 80000
HALF_PAD = 81920
E_PAD = 2 * HALF_PAD
PAD = HALF_PAD - HALF

NC, NS = 2, 16
NW = NC * NS
PER_W = E_PAD // NW
CHUNK = 128
N_CHUNKS = PER_W // CHUNK
SUP = 256
SPC = SUP // CHUNK
N_SUP = PER_W // SUP
SUP_S = CHUNK
N_SUP_S = PER_W // SUP_S

N_A_PAD = 10240
ROWS_PER_SUB = N_A_PAD // NS
ZCHUNK = 64

BLK = 2048
NBLK = E_PAD // BLK
BLKA = 1000
NBLKA = N_ATOMS // BLKA

_HIGHEST = lax.Precision.HIGHEST


def _worker_id():
    return lax.axis_index("c") * NS + lax.axis_index("s")


_sc_cache = {}


def _sc_kernels():
    if _sc_cache:
        return _sc_cache["g1"], _sc_cache["gsum"], _sc_cache["scat"]

    mesh = plsc.VectorSubcoreMesh(core_axis_name="c", subcore_axis_name="s",
                                  num_cores=NC, num_subcores=NS)

    @functools.partial(
        pl.kernel,
        out_type=jax.ShapeDtypeStruct((E_PAD, H), jnp.float32),
        mesh=mesh,
        scratch_types=[
            pltpu.VMEM((N_CHUNKS, CHUNK), jnp.int32),
            pltpu.VMEM((CHUNK, H), jnp.float32),
            pltpu.VMEM((CHUNK, H), jnp.float32),
            pltpu.VMEM_SHARED((N_A_PAD, H), jnp.float32),
            pltpu.SemaphoreType.DMA,
            pltpu.SemaphoreType.DMA,
            pltpu.SemaphoreType.DMA,
            pltpu.SemaphoreType.DMA,
        ],
    )
    def _sc_gather1(tab_hbm, idx_hbm, out_hbm, idx_v, b0, b1, tab_sh,
                    gs0, gs1, os0, os1):
        s = lax.axis_index("s")
        wid = lax.axis_index("c") * NS + lax.axis_index("s")
        base = wid * PER_W
        pltpu.sync_copy(idx_hbm.at[wid], idx_v)

        def stage(k, carry):
            r = s * ROWS_PER_SUB + k * CHUNK
            pltpu.sync_copy(tab_hbm.at[pl.ds(r, CHUNK)], b0)
            pltpu.sync_copy(b0, tab_sh.at[pl.ds(r, CHUNK)])
            return carry

        lax.fori_loop(0, ROWS_PER_SUB // CHUNK, stage, None)
        plsc.subcore_barrier()

        def fire_g(j, buf, sem):
            pltpu.async_copy(tab_sh.at[idx_v.at[j]], buf, sem)

        def drain_g(buf, sem):
            pltpu.make_async_copy(tab_sh.at[idx_v.at[0]], buf, sem).wait()

        def fire_out(j, buf, sem):
            pltpu.async_copy(buf, out_hbm.at[pl.ds(base + j * CHUNK, CHUNK)],
                             sem)

        def wait_out(buf, sem):
            pltpu.make_async_copy(buf, out_hbm.at[pl.ds(base, CHUNK)],
                                  sem).wait()

        fire_g(0, b0, gs0)

        def body(jj, carry):
            j0 = 2 * jj
            drain_g(b0, gs0)

            @pl.when(jj > 0)
            def _():
                wait_out(b1, os1)

            fire_g(j0 + 1, b1, gs1)
            fire_out(j0, b0, os0)
            drain_g(b1, gs1)
            wait_out(b0, os0)

            @pl.when(jj < N_CHUNKS // 2 - 1)
            def _():
                fire_g(j0 + 2, b0, gs0)

            fire_out(j0 + 1, b1, os1)
            return carry

        lax.fori_loop(0, N_CHUNKS // 2, body, None)
        wait_out(b1, os1)

    @functools.partial(
        pl.kernel,
        out_type=jax.ShapeDtypeStruct((E_PAD, H), jnp.float32),
        mesh=mesh,
        scratch_types=[
            pltpu.VMEM((N_CHUNKS, CHUNK), jnp.int32),
            pltpu.VMEM((CHUNK, H), jnp.float32),
            pltpu.VMEM((CHUNK, H), jnp.float32),
            pltpu.VMEM_SHARED((N_A_PAD, H), jnp.float32),
            pltpu.SemaphoreType.DMA,
            pltpu.SemaphoreType.DMA,
            pltpu.SemaphoreType.DMA,
            pltpu.SemaphoreType.DMA,
        ],
    )
    def _sc_gather_sum(tab2_hbm, idx_hbm, out_hbm, idx_v, b0, b1, tab_sh,
                       gs0, gs1, os0, os1):
        s = lax.axis_index("s")
        wid = lax.axis_index("c") * NS + lax.axis_index("s")
        base = wid * PER_W
        pltpu.sync_copy(idx_hbm.at[wid], idx_v)

        if True:
            def stage(k, carry):
                r = s * ROWS_PER_SUB + k * CHUNK
                pltpu.sync_copy(tab2_hbm.at[pl.ds(r, CHUNK)], b0)
                pltpu.sync_copy(tab2_hbm.at[pl.ds(N_A_PAD + r, CHUNK)], b1)

                def add_row(i, carry2):
                    for cc in range(H // 16):
                        sl = pl.ds(cc * 16, 16)
                        b0[i, sl] = b0[i, sl] + b1[i, sl]
                    return carry2

                lax.fori_loop(0, CHUNK, add_row, None)
                pltpu.sync_copy(b0, tab_sh.at[pl.ds(r, CHUNK)])
                return carry

            lax.fori_loop(0, ROWS_PER_SUB // CHUNK, stage, None)
            plsc.subcore_barrier()

            def fire_g(j, buf, sem):
                pltpu.async_copy(tab_sh.at[idx_v.at[j]], buf, sem)

            def drain_g(buf, sem):
                pltpu.make_async_copy(tab_sh.at[idx_v.at[0]], buf, sem).wait()

            def fire_out(j, buf, sem):
                pltpu.async_copy(buf,
                                 out_hbm.at[pl.ds(base + j * CHUNK, CHUNK)],
                                 sem)

            def wait_out(buf, sem):
                pltpu.make_async_copy(buf, out_hbm.at[pl.ds(base, CHUNK)],
                                      sem).wait()

            fire_g(0, b0, gs0)

            def body(jj, carry):
                j0 = 2 * jj
                drain_g(b0, gs0)

                @pl.when(jj > 0)
                def _():
                    wait_out(b1, os1)

                fire_g(j0 + 1, b1, gs1)
                fire_out(j0, b0, os0)
                drain_g(b1, gs1)
                wait_out(b0, os0)

                @pl.when(jj < N_CHUNKS // 2 - 1)
                def _():
                    fire_g(j0 + 2, b0, gs0)

                fire_out(j0 + 1, b1, os1)
                return carry

            lax.fori_loop(0, N_CHUNKS // 2, body, None)
            wait_out(b1, os1)


    @functools.partial(
        pl.kernel,
        out_type=jax.ShapeDtypeStruct((2 * N_A_PAD, H), jnp.float32),
        mesh=mesh,
        scratch_types=[
            pltpu.VMEM((N_CHUNKS, CHUNK), jnp.int32),
            pltpu.VMEM((SUP_S, H), jnp.float32),
            pltpu.VMEM((SUP_S, H), jnp.float32),
            pltpu.VMEM((ZCHUNK, H), jnp.float32),
            pltpu.VMEM_SHARED((N_A_PAD, H), jnp.float32),
            pltpu.SemaphoreType.DMA,
            pltpu.SemaphoreType.DMA,
            pltpu.SemaphoreType.DMA,
            pltpu.SemaphoreType.DMA,
        ],
    )
    def _sc_scatter(rows_hbm, idx_hbm, out_hbm, idx_v, b0, b1, zbuf_v,
                    agg_sh, ls0, ls1, ss0, ss1):
        c = lax.axis_index("c")
        s = lax.axis_index("s")
        wid = c * NS + s
        base = wid * PER_W

        def zrow(i, carry):
            for cc in range(H // 16):
                zbuf_v[i, pl.ds(cc * 16, 16)] = jnp.zeros((16,), jnp.float32)
            return carry

        lax.fori_loop(0, ZCHUNK, zrow, None)

        def zcp(k, carry):
            r = s * ROWS_PER_SUB + k * ZCHUNK
            pltpu.sync_copy(zbuf_v, agg_sh.at[pl.ds(r, ZCHUNK)])
            return carry

        lax.fori_loop(0, ROWS_PER_SUB // ZCHUNK, zcp, None)
        plsc.subcore_barrier()

        pltpu.sync_copy(idx_hbm.at[wid], idx_v)

        def fire_load(j, buf, sem):
            pltpu.async_copy(rows_hbm.at[pl.ds(base + j * SUP_S, SUP_S)],
                             buf, sem)

        def wait_load(buf, sem):
            pltpu.make_async_copy(rows_hbm.at[pl.ds(base, SUP_S)], buf,
                                  sem).wait()

        def fire_add(j, buf, sem):
            pltpu.async_copy(buf, agg_sh.at[idx_v.at[j]], sem, add=True)

        def drain_add(buf, sem):
            pltpu.make_async_copy(buf, agg_sh.at[idx_v.at[0]], sem).wait()

        fire_load(0, b0, ls0)

        def sup(jj, carry):
            j0 = 2 * jj
            wait_load(b0, ls0)
            fire_add(j0, b0, ss0)

            @pl.when(jj > 0)
            def _():
                drain_add(b1, ss1)

            fire_load(j0 + 1, b1, ls1)
            drain_add(b0, ss0)

            @pl.when(jj < N_SUP_S // 2 - 1)
            def _():
                fire_load(j0 + 2, b0, ls0)

            wait_load(b1, ls1)
            fire_add(j0 + 1, b1, ss1)
            return carry

        lax.fori_loop(0, N_SUP_S // 2, sup, None)
        drain_add(b1, ss1)
        plsc.subcore_barrier()

        def cp(k, carry):
            r = s * ROWS_PER_SUB + k * ZCHUNK
            pltpu.sync_copy(agg_sh.at[pl.ds(r, ZCHUNK)], zbuf_v)
            pltpu.sync_copy(zbuf_v, out_hbm.at[pl.ds(c * N_A_PAD + r, ZCHUNK)])
            return carry

        lax.fori_loop(0, ROWS_PER_SUB // ZCHUNK, cp, None)

    _sc_cache.update(g1=_sc_gather1, gsum=_sc_gather_sum, scat=_sc_scatter)
    return _sc_cache["g1"], _sc_cache["gsum"], _sc_cache["scat"]


def _pad_mask(i, blk):
    row = i * blk + lax.broadcasted_iota(jnp.int32, (blk, H), 0)
    return (row % HALF_PAD) < HALF


def _init_body(bond_ref, xg_ref, wb_ref, wx_ref, b_ref, out_ref):
    acc = jnp.dot(bond_ref[...], wb_ref[...],
                  preferred_element_type=jnp.float32, precision=_HIGHEST)
    acc += jnp.dot(xg_ref[...], wx_ref[...],
                   preferred_element_type=jnp.float32, precision=_HIGHEST)
    h0 = jnp.maximum(acc + b_ref[...], 0.0)
    out_ref[...] = jnp.where(_pad_mask(pl.program_id(0), BLK), h0, 0.0)


_tc_init = pl.pallas_call(
    _init_body,
    grid=(NBLK,),
    in_specs=[
        pl.BlockSpec((BLK, BOND_DIM), lambda i: (i, 0)),
        pl.BlockSpec((BLK, H), lambda i: (i, 0)),
        pl.BlockSpec((BOND_DIM, H), lambda i: (0, 0)),
        pl.BlockSpec((H, H), lambda i: (0, 0)),
        pl.BlockSpec((1, H), lambda i: (0, 0)),
    ],
    out_specs=pl.BlockSpec((BLK, H), lambda i: (i, 0)),
    out_shape=jax.ShapeDtypeStruct((E_PAD, H), jnp.float32),
)


def _depth_body(h0_ref, g_ref, hrev_ref, w_ref, b_ref, out_ref):
    m = g_ref[...] - hrev_ref[...]
    acc = jnp.dot(m, w_ref[...],
                  preferred_element_type=jnp.float32, precision=_HIGHEST)
    h = jnp.maximum(h0_ref[...] + acc + b_ref[...], 0.0)
    out_ref[...] = jnp.where(_pad_mask(pl.program_id(0), BLK), h, 0.0)


_tc_depth = pl.pallas_call(
    _depth_body,
    grid=(NBLK,),
    in_specs=[
        pl.BlockSpec((BLK, H), lambda i: (i, 0)),
        pl.BlockSpec((BLK, H), lambda i: (i, 0)),
        pl.BlockSpec((BLK, H), lambda i: ((i + NBLK // 2) % NBLK, 0)),
        pl.BlockSpec((H, H), lambda i: (0, 0)),
        pl.BlockSpec((1, H), lambda i: (0, 0)),
    ],
    out_specs=pl.BlockSpec((BLK, H), lambda i: (i, 0)),
    out_shape=jax.ShapeDtypeStruct((E_PAD, H), jnp.float32),
)


def _final_body(atom_ref, agga_ref, aggb_ref, a2m_ref, molf_ref,
                wx_ref, wm_ref, b_ref, out_ref):
    i = pl.program_id(0)
    mv = agga_ref[...] + aggb_ref[...]
    hv = jnp.dot(atom_ref[...], wx_ref[...],
                 preferred_element_type=jnp.float32, precision=_HIGHEST)
    hv += jnp.dot(mv, wm_ref[...],
                  preferred_element_type=jnp.float32, precision=_HIGHEST)
    hv = jnp.maximum(hv + b_ref[...], 0.0)
    seg = a2m_ref[0, 0, :]
    onehot = (lax.broadcasted_iota(jnp.int32, (N_MOL, BLKA), 0)
              == seg[None, :]).astype(jnp.float32)
    contrib = jnp.dot(onehot, hv,
                      preferred_element_type=jnp.float32, precision=_HIGHEST)

    @pl.when(i == 0)
    def _():
        out_ref[:, :H] = contrib
        out_ref[:, H:] = molf_ref[...]

    @pl.when(i != 0)
    def _():
        out_ref[:, :H] = out_ref[:, :H] + contrib


_tc_final = pl.pallas_call(
    _final_body,
    grid=(NBLKA,),
    in_specs=[
        pl.BlockSpec((BLKA, H), lambda i: (i, 0)),
        pl.BlockSpec((BLKA, H), lambda i: (i, 0)),
        pl.BlockSpec((BLKA, H), lambda i: (i, 0)),
        pl.BlockSpec((1, 1, BLKA), lambda i: (i, 0, 0)),
        pl.BlockSpec((N_MOL, H), lambda i: (0, 0)),
        pl.BlockSpec((H, H), lambda i: (0, 0)),
        pl.BlockSpec((H, H), lambda i: (0, 0)),
        pl.BlockSpec((1, H), lambda i: (0, 0)),
    ],
    out_specs=pl.BlockSpec((N_MOL, 2 * H), lambda i: (0, 0)),
    out_shape=jax.ShapeDtypeStruct((N_MOL, 2 * H), jnp.float32),
)


def _mid_pad(x):
    z = jnp.zeros((PAD,) + x.shape[1:], dtype=x.dtype)
    return jnp.concatenate([x[:HALF], z, x[HALF:], z], axis=0)


def _mid_pad_idx(x):
    z = jnp.arange(PAD, dtype=x.dtype) % N_ATOMS
    return jnp.concatenate([x[:HALF], z, x[HALF:], z], axis=0)


def kernel(atom_features, bond_features, molecule_features, Wi_w, Wi_b,
           Wm_w, Wm_b, Wa_w, Wa_b, bond_index, atom_to_molecule, b2rev):
    src = bond_index[0].astype(jnp.int32)
    dst = bond_index[1].astype(jnp.int32)

    src_p = _mid_pad_idx(src)
    dst_p = _mid_pad_idx(dst)
    bond_p = _mid_pad(bond_features)

    src_t = src_p.reshape(NW, N_CHUNKS, CHUNK)
    dst_t = dst_p.reshape(NW, N_CHUNKS, CHUNK)

    wi_t = Wi_w.T
    wb_t = wi_t[:BOND_DIM]
    wx_t = wi_t[BOND_DIM:]
    wm_t = Wm_w.T
    wa_t = Wa_w.T
    wax_t = wa_t[:H]
    wam_t = wa_t[H:]

    wi_b = Wi_b.reshape(1, H)
    wm_b = Wm_b.reshape(1, H)
    wa_b = Wa_b.reshape(1, H)

    a2m_t = atom_to_molecule.astype(jnp.int32).reshape(NBLKA, 1, BLKA)

    sc_gather1, sc_gather_sum, sc_scatter = _sc_kernels()

    xg = sc_gather1(atom_features, src_t)
    h0 = _tc_init(bond_p, xg, wb_t, wx_t, wi_b)

    h = h0
    for _ in range(DEPTH):
        agg2 = sc_scatter(h, dst_t)
        g = sc_gather_sum(agg2, src_t)
        h = _tc_depth(h0, g, h, wm_t, wm_b)

    agg2f = sc_scatter(h, src_t)
    return _tc_final(atom_features, agg2f[:N_ATOMS], agg2f[N_A_PAD:N_A_PAD + N_ATOMS],
                     a2m_t, molecule_features, wax_t, wam_t, wa_b)


# speedup vs baseline: 1.0679x; 1.0021x over previous
import functools

import jax
import jax.numpy as jnp
from jax import lax
from jax.experimental import pallas as pl
from jax.experimental.pallas import tpu as pltpu
from jax.experimental.pallas import tpu_sc as plsc

N_ATOMS = 10000
N_MOL = 500
H = 128
BOND_DIM = 16
DEPTH = 3

HALF = 80000
HALF_PAD = 81920
E_PAD = 2 * HALF_PAD
PAD = HALF_PAD - HALF

NC, NS = 2, 16
NW = NC * NS
PER_W = E_PAD // NW
CHUNK = 128
N_CHUNKS = PER_W // CHUNK
SUP = 256
SPC = SUP // CHUNK
N_SUP = PER_W // SUP
SUP_S = CHUNK
N_SUP_S = PER_W // SUP_S

N_A_PAD = 10240
ROWS_PER_SUB = N_A_PAD // NS
ZCHUNK = 64

BLK = 2048
NBLK = E_PAD // BLK
BLKA = 1000
NBLKA = N_ATOMS // BLKA

_HIGHEST = lax.Precision.HIGHEST


_sc_cache = {}


def _sc_kernels():
    if _sc_cache:
        return _sc_cache["g1"], _sc_cache["gsum"], _sc_cache["scat"]

    mesh = plsc.VectorSubcoreMesh(core_axis_name="c", subcore_axis_name="s",
                                  num_cores=NC, num_subcores=NS)

    @functools.partial(
        pl.kernel,
        out_type=jax.ShapeDtypeStruct((E_PAD, H), jnp.float32),
        mesh=mesh,
        scratch_types=[
            pltpu.VMEM((N_CHUNKS, CHUNK), jnp.int32),
            pltpu.VMEM((CHUNK, H), jnp.float32),
            pltpu.VMEM((CHUNK, H), jnp.float32),
            pltpu.VMEM_SHARED((N_A_PAD, H), jnp.float32),
            pltpu.SemaphoreType.DMA,
            pltpu.SemaphoreType.DMA,
            pltpu.SemaphoreType.DMA,
            pltpu.SemaphoreType.DMA,
        ],
    )
    def _sc_gather1(tab_hbm, idx_hbm, out_hbm, idx_v, b0, b1, tab_sh,
                    gs0, gs1, os0, os1):
        s = lax.axis_index("s")
        wid = lax.axis_index("c") * NS + lax.axis_index("s")
        base = wid * PER_W
        pltpu.sync_copy(idx_hbm.at[wid], idx_v)

        def stage(k, carry):
            r = s * ROWS_PER_SUB + k * CHUNK
            pltpu.sync_copy(tab_hbm.at[pl.ds(r, CHUNK)], b0)
            pltpu.sync_copy(b0, tab_sh.at[pl.ds(r, CHUNK)])
            return carry

        lax.fori_loop(0, ROWS_PER_SUB // CHUNK, stage, None)
        plsc.subcore_barrier()

        def fire_g(j, buf, sem):
            pltpu.async_copy(tab_sh.at[idx_v.at[j]], buf, sem)

        def drain_g(buf, sem):
            pltpu.make_async_copy(tab_sh.at[idx_v.at[0]], buf, sem).wait()

        def fire_out(j, buf, sem):
            pltpu.async_copy(buf, out_hbm.at[pl.ds(base + j * CHUNK, CHUNK)],
                             sem)

        def wait_out(buf, sem):
            pltpu.make_async_copy(buf, out_hbm.at[pl.ds(base, CHUNK)],
                                  sem).wait()

        fire_g(0, b0, gs0)

        def body(jj, carry):
            j0 = 2 * jj
            drain_g(b0, gs0)

            @pl.when(jj > 0)
            def _():
                wait_out(b1, os1)

            fire_g(j0 + 1, b1, gs1)
            fire_out(j0, b0, os0)
            drain_g(b1, gs1)
            wait_out(b0, os0)

            @pl.when(jj < N_CHUNKS // 2 - 1)
            def _():
                fire_g(j0 + 2, b0, gs0)

            fire_out(j0 + 1, b1, os1)
            return carry

        lax.fori_loop(0, N_CHUNKS // 2, body, None)
        wait_out(b1, os1)

    @functools.partial(
        pl.kernel,
        out_type=jax.ShapeDtypeStruct((E_PAD, H), jnp.float32),
        mesh=mesh,
        scratch_types=[
            pltpu.VMEM((N_CHUNKS, CHUNK), jnp.int32),
            pltpu.VMEM((CHUNK, H), jnp.float32),
            pltpu.VMEM((CHUNK, H), jnp.float32),
            pltpu.VMEM_SHARED((N_A_PAD, H), jnp.float32),
            pltpu.SemaphoreType.DMA,
            pltpu.SemaphoreType.DMA,
            pltpu.SemaphoreType.DMA,
            pltpu.SemaphoreType.DMA,
        ],
    )
    def _sc_gather_sum(tab2_hbm, idx_hbm, out_hbm, idx_v, b0, b1, tab_sh,
                       gs0, gs1, os0, os1):
        s = lax.axis_index("s")
        wid = lax.axis_index("c") * NS + lax.axis_index("s")
        base = wid * PER_W
        pltpu.sync_copy(idx_hbm.at[wid], idx_v)

        if True:
            def stage(k, carry):
                r = s * ROWS_PER_SUB + k * CHUNK
                pltpu.sync_copy(tab2_hbm.at[pl.ds(r, CHUNK)], b0)
                pltpu.sync_copy(tab2_hbm.at[pl.ds(N_A_PAD + r, CHUNK)], b1)

                def add_row(i, carry2):
                    for cc in range(H // 16):
                        sl = pl.ds(cc * 16, 16)
                        b0[i, sl] = b0[i, sl] + b1[i, sl]
                    return carry2

                lax.fori_loop(0, CHUNK, add_row, None)
                pltpu.sync_copy(b0, tab_sh.at[pl.ds(r, CHUNK)])
                return carry

            lax.fori_loop(0, ROWS_PER_SUB // CHUNK, stage, None)
            plsc.subcore_barrier()

            def fire_g(j, buf, sem):
                pltpu.async_copy(tab_sh.at[idx_v.at[j]], buf, sem)

            def drain_g(buf, sem):
                pltpu.make_async_copy(tab_sh.at[idx_v.at[0]], buf, sem).wait()

            def fire_out(j, buf, sem):
                pltpu.async_copy(buf,
                                 out_hbm.at[pl.ds(base + j * CHUNK, CHUNK)],
                                 sem)

            def wait_out(buf, sem):
                pltpu.make_async_copy(buf, out_hbm.at[pl.ds(base, CHUNK)],
                                      sem).wait()

            fire_g(0, b0, gs0)

            def body(jj, carry):
                j0 = 2 * jj
                drain_g(b0, gs0)

                @pl.when(jj > 0)
                def _():
                    wait_out(b1, os1)

                fire_g(j0 + 1, b1, gs1)
                fire_out(j0, b0, os0)
                drain_g(b1, gs1)
                wait_out(b0, os0)

                @pl.when(jj < N_CHUNKS // 2 - 1)
                def _():
                    fire_g(j0 + 2, b0, gs0)

                fire_out(j0 + 1, b1, os1)
                return carry

            lax.fori_loop(0, N_CHUNKS // 2, body, None)
            wait_out(b1, os1)


    @functools.partial(
        pl.kernel,
        out_type=jax.ShapeDtypeStruct((2 * N_A_PAD, H), jnp.float32),
        mesh=mesh,
        scratch_types=[
            pltpu.VMEM((N_CHUNKS, CHUNK), jnp.int32),
            pltpu.VMEM((SUP_S, H), jnp.float32),
            pltpu.VMEM((SUP_S, H), jnp.float32),
            pltpu.VMEM((ZCHUNK, H), jnp.float32),
            pltpu.VMEM_SHARED((N_A_PAD, H), jnp.float32),
            pltpu.SemaphoreType.DMA,
            pltpu.SemaphoreType.DMA,
            pltpu.SemaphoreType.DMA,
            pltpu.SemaphoreType.DMA,
        ],
    )
    def _sc_scatter(rows_hbm, idx_hbm, out_hbm, idx_v, b0, b1, zbuf_v,
                    agg_sh, ls0, ls1, ss0, ss1):
        c = lax.axis_index("c")
        s = lax.axis_index("s")
        wid = c * NS + s
        base = wid * PER_W

        def zrow(i, carry):
            for cc in range(H // 16):
                zbuf_v[i, pl.ds(cc * 16, 16)] = jnp.zeros((16,), jnp.float32)
            return carry

        lax.fori_loop(0, ZCHUNK, zrow, None)

        def zcp(k, carry):
            r = s * ROWS_PER_SUB + k * ZCHUNK
            pltpu.sync_copy(zbuf_v, agg_sh.at[pl.ds(r, ZCHUNK)])
            return carry

        lax.fori_loop(0, ROWS_PER_SUB // ZCHUNK, zcp, None)
        plsc.subcore_barrier()

        pltpu.sync_copy(idx_hbm.at[wid], idx_v)

        def fire_load(j, buf, sem):
            pltpu.async_copy(rows_hbm.at[pl.ds(base + j * SUP_S, SUP_S)],
                             buf, sem)

        def wait_load(buf, sem):
            pltpu.make_async_copy(rows_hbm.at[pl.ds(base, SUP_S)], buf,
                                  sem).wait()

        def fire_add(j, buf, sem):
            pltpu.async_copy(buf, agg_sh.at[idx_v.at[j]], sem, add=True)

        def drain_add(buf, sem):
            pltpu.make_async_copy(buf, agg_sh.at[idx_v.at[0]], sem).wait()

        fire_load(0, b0, ls0)

        def sup(jj, carry):
            j0 = 2 * jj
            wait_load(b0, ls0)
            fire_add(j0, b0, ss0)

            @pl.when(jj > 0)
            def _():
                drain_add(b1, ss1)

            fire_load(j0 + 1, b1, ls1)
            drain_add(b0, ss0)

            @pl.when(jj < N_SUP_S // 2 - 1)
            def _():
                fire_load(j0 + 2, b0, ls0)

            wait_load(b1, ls1)
            fire_add(j0 + 1, b1, ss1)
            return carry

        lax.fori_loop(0, N_SUP_S // 2, sup, None)
        drain_add(b1, ss1)
        plsc.subcore_barrier()

        def cp(k, carry):
            r = s * ROWS_PER_SUB + k * ZCHUNK
            pltpu.sync_copy(agg_sh.at[pl.ds(r, ZCHUNK)], zbuf_v)
            pltpu.sync_copy(zbuf_v, out_hbm.at[pl.ds(c * N_A_PAD + r, ZCHUNK)])
            return carry

        lax.fori_loop(0, ROWS_PER_SUB // ZCHUNK, cp, None)

    _sc_cache.update(g1=_sc_gather1, gsum=_sc_gather_sum, scat=_sc_scatter)
    return _sc_cache["g1"], _sc_cache["gsum"], _sc_cache["scat"]


def _pad_mask(i, blk):
    row = i * blk + lax.broadcasted_iota(jnp.int32, (blk, H), 0)
    return (row % HALF_PAD) < HALF


def _init_body(bond_ref, xg_ref, wb_ref, wx_ref, b_ref, out_ref):
    acc = jnp.dot(bond_ref[...], wb_ref[...],
                  preferred_element_type=jnp.float32, precision=_HIGHEST)
    acc += jnp.dot(xg_ref[...], wx_ref[...],
                   preferred_element_type=jnp.float32, precision=_HIGHEST)
    h0 = jnp.maximum(acc + b_ref[...], 0.0)
    out_ref[...] = jnp.where(_pad_mask(pl.program_id(0), BLK), h0, 0.0)


_tc_init = pl.pallas_call(
    _init_body,
    grid=(NBLK,),
    in_specs=[
        pl.BlockSpec((BLK, BOND_DIM), lambda i: (i, 0)),
        pl.BlockSpec((BLK, H), lambda i: (i, 0)),
        pl.BlockSpec((BOND_DIM, H), lambda i: (0, 0)),
        pl.BlockSpec((H, H), lambda i: (0, 0)),
        pl.BlockSpec((1, H), lambda i: (0, 0)),
    ],
    out_specs=pl.BlockSpec((BLK, H), lambda i: (i, 0)),
    out_shape=jax.ShapeDtypeStruct((E_PAD, H), jnp.float32),
)


def _depth_body(h0_ref, g_ref, hrev_ref, w_ref, b_ref, out_ref):
    m = g_ref[...] - hrev_ref[...]
    acc = jnp.dot(m, w_ref[...],
                  preferred_element_type=jnp.float32, precision=_HIGHEST)
    h = jnp.maximum(h0_ref[...] + acc + b_ref[...], 0.0)
    out_ref[...] = jnp.where(_pad_mask(pl.program_id(0), BLK), h, 0.0)


_tc_depth = pl.pallas_call(
    _depth_body,
    grid=(NBLK,),
    in_specs=[
        pl.BlockSpec((BLK, H), lambda i: (i, 0)),
        pl.BlockSpec((BLK, H), lambda i: (i, 0)),
        pl.BlockSpec((BLK, H), lambda i: ((i + NBLK // 2) % NBLK, 0)),
        pl.BlockSpec((H, H), lambda i: (0, 0)),
        pl.BlockSpec((1, H), lambda i: (0, 0)),
    ],
    out_specs=pl.BlockSpec((BLK, H), lambda i: (i, 0)),
    out_shape=jax.ShapeDtypeStruct((E_PAD, H), jnp.float32),
)


def _final_body(atom_ref, agga_ref, aggb_ref, a2m_ref, molf_ref,
                wx_ref, wm_ref, b_ref, out_ref):
    i = pl.program_id(0)
    mv = agga_ref[...] + aggb_ref[...]
    hv = jnp.dot(atom_ref[...], wx_ref[...],
                 preferred_element_type=jnp.float32, precision=_HIGHEST)
    hv += jnp.dot(mv, wm_ref[...],
                  preferred_element_type=jnp.float32, precision=_HIGHEST)
    hv = jnp.maximum(hv + b_ref[...], 0.0)
    seg = a2m_ref[0, 0, :]
    onehot = (lax.broadcasted_iota(jnp.int32, (N_MOL, BLKA), 0)
              == seg[None, :]).astype(jnp.float32)
    contrib = jnp.dot(onehot, hv,
                      preferred_element_type=jnp.float32, precision=_HIGHEST)

    @pl.when(i == 0)
    def _():
        out_ref[:, :H] = contrib
        out_ref[:, H:] = molf_ref[...]

    @pl.when(i != 0)
    def _():
        out_ref[:, :H] = out_ref[:, :H] + contrib


_tc_final = pl.pallas_call(
    _final_body,
    grid=(NBLKA,),
    in_specs=[
        pl.BlockSpec((BLKA, H), lambda i: (i, 0)),
        pl.BlockSpec((BLKA, H), lambda i: (i, 0)),
        pl.BlockSpec((BLKA, H), lambda i: (i, 0)),
        pl.BlockSpec((1, 1, BLKA), lambda i: (i, 0, 0)),
        pl.BlockSpec((N_MOL, H), lambda i: (0, 0)),
        pl.BlockSpec((H, H), lambda i: (0, 0)),
        pl.BlockSpec((H, H), lambda i: (0, 0)),
        pl.BlockSpec((1, H), lambda i: (0, 0)),
    ],
    out_specs=pl.BlockSpec((N_MOL, 2 * H), lambda i: (0, 0)),
    out_shape=jax.ShapeDtypeStruct((N_MOL, 2 * H), jnp.float32),
)


def _mid_pad(x):
    z = jnp.zeros((PAD,) + x.shape[1:], dtype=x.dtype)
    return jnp.concatenate([x[:HALF], z, x[HALF:], z], axis=0)


def _mid_pad_idx(x):
    z = jnp.arange(PAD, dtype=x.dtype) % N_ATOMS
    return jnp.concatenate([x[:HALF], z, x[HALF:], z], axis=0)


def kernel(atom_features, bond_features, molecule_features, Wi_w, Wi_b,
           Wm_w, Wm_b, Wa_w, Wa_b, bond_index, atom_to_molecule, b2rev):
    src = bond_index[0].astype(jnp.int32)
    dst = bond_index[1].astype(jnp.int32)

    src_p = _mid_pad_idx(src)
    dst_p = _mid_pad_idx(dst)
    bond_p = _mid_pad(bond_features)

    src_t = src_p.reshape(NW, N_CHUNKS, CHUNK)
    dst_t = dst_p.reshape(NW, N_CHUNKS, CHUNK)

    wi_t = Wi_w.T
    wb_t = wi_t[:BOND_DIM]
    wx_t = wi_t[BOND_DIM:]
    wm_t = Wm_w.T
    wa_t = Wa_w.T
    wax_t = wa_t[:H]
    wam_t = wa_t[H:]

    wi_b = Wi_b.reshape(1, H)
    wm_b = Wm_b.reshape(1, H)
    wa_b = Wa_b.reshape(1, H)

    a2m_t = atom_to_molecule.astype(jnp.int32).reshape(NBLKA, 1, BLKA)

    sc_gather1, sc_gather_sum, sc_scatter = _sc_kernels()

    xg = sc_gather1(atom_features, src_t)
    h0 = _tc_init(bond_p, xg, wb_t, wx_t, wi_b)

    h = h0
    for _ in range(DEPTH):
        agg2 = sc_scatter(h, dst_t)
        g = sc_gather_sum(agg2, src_t)
        h = _tc_depth(h0, g, h, wm_t, wm_b)

    agg2f = sc_scatter(h, src_t)
    return _tc_final(atom_features, agg2f[:N_ATOMS], agg2f[N_A_PAD:N_A_PAD + N_ATOMS],
                     a2m_t, molecule_features, wax_t, wam_t, wa_b)
